# baseline (speedup 1.0000x reference)
import jax
import jax.numpy as jnp
from jax.experimental import pallas as pl
from jax.experimental.pallas import tpu as pltpu

_B = 256


def _rot_body(pe_hbm, o_ref, seed_v, sw_v, cs_v, step_v, sem):
    b = pl.program_id(0)
    d = o_ref.shape[-1]

    @pl.when(b == 0)
    def _init():
        cp = pltpu.make_async_copy(pe_hbm.at[pl.ds(0, _B)], seed_v, sem)
        cp.start()
        cp.wait()
        seed = seed_v[...]
        lane = jax.lax.broadcasted_iota(jnp.int32, (_B, d), 1)
        even = (lane % 2) == 0
        sw_v[...] = jnp.where(
            even, jnp.roll(seed, -1, axis=1), -jnp.roll(seed, 1, axis=1)
        )
        row = seed_v[_B - 1 : _B, :]
        lane1 = jax.lax.broadcasted_iota(jnp.int32, (1, d), 1)
        even1 = (lane1 % 2) == 0
        cstep = jnp.where(even1, jnp.roll(row, -1, axis=1), row)
        sstep = jnp.where(even1, row, jnp.roll(row, 1, axis=1))
        step_v[0:1, :] = cstep
        step_v[1:2, :] = sstep
        cs_v[0:1, :] = jnp.ones((1, d), jnp.float32)
        cs_v[1:2, :] = jnp.zeros((1, d), jnp.float32)

    c = cs_v[0:1, :]
    s = cs_v[1:2, :]
    o_ref[...] = seed_v[...] * c + sw_v[...] * s
    cstep = step_v[0:1, :]
    sstep = step_v[1:2, :]
    cs_v[0:1, :] = c * cstep - s * sstep
    cs_v[1:2, :] = s * cstep + c * sstep


def kernel(x, pe):
    seq_len = x.shape[-1]
    d = pe.shape[-1]
    nb = seq_len // _B

    out = pl.pallas_call(
        _rot_body,
        grid=(nb,),
        in_specs=[pl.BlockSpec(memory_space=pl.ANY)],
        out_specs=pl.BlockSpec((_B, d), lambda i: (i, 0)),
        out_shape=jax.ShapeDtypeStruct((seq_len, d), jnp.float32),
        scratch_shapes=[
            pltpu.VMEM((_B, d), jnp.float32),
            pltpu.VMEM((_B, d), jnp.float32),
            pltpu.VMEM((2, d), jnp.float32),
            pltpu.VMEM((2, d), jnp.float32),
            pltpu.SemaphoreType.DMA,
        ],
    )(pe)
    return out[None]


# speedup vs baseline: 1.2458x; 1.2458x over previous
import jax
import jax.numpy as jnp
from jax.experimental import pallas as pl
from jax.experimental.pallas import tpu as pltpu

_B = 128
_NBUF = 4


def _rot_body(pe_hbm, o_hbm, seed_v, sw_v, cs_v, step_v, obuf, osem, isem):
    b = pl.program_id(0)
    nb = pl.num_programs(0)
    d = o_hbm.shape[-1]

    @pl.when(b == 0)
    def _init():
        cp = pltpu.make_async_copy(pe_hbm.at[pl.ds(0, _B)], seed_v, isem)
        cp.start()
        cp.wait()
        seed = seed_v[...]
        lane = jax.lax.broadcasted_iota(jnp.int32, (_B, d), 1)
        even = (lane % 2) == 0
        sw_v[...] = jnp.where(
            even, jnp.roll(seed, -1, axis=1), -jnp.roll(seed, 1, axis=1)
        )
        row = seed_v[_B - 1 : _B, :]
        lane1 = jax.lax.broadcasted_iota(jnp.int32, (1, d), 1)
        even1 = (lane1 % 2) == 0
        step_v[0:1, :] = jnp.where(even1, jnp.roll(row, -1, axis=1), row)
        step_v[1:2, :] = jnp.where(even1, row, jnp.roll(row, 1, axis=1))
        cs_v[0:1, :] = jnp.ones((1, d), jnp.float32)
        cs_v[1:2, :] = jnp.zeros((1, d), jnp.float32)

    @pl.when(b >= _NBUF)
    def _credit():
        pltpu.make_async_copy(
            obuf.at[pl.ds(0, _B)], o_hbm.at[pl.ds(0, _B)], osem
        ).wait()

    slot = jax.lax.rem(b, _NBUF)
    c = cs_v[0:1, :]
    s = cs_v[1:2, :]
    obuf[pl.ds(slot * _B, _B), :] = seed_v[...] * c + sw_v[...] * s
    pltpu.make_async_copy(
        obuf.at[pl.ds(slot * _B, _B)], o_hbm.at[pl.ds(b * _B, _B)], osem
    ).start()
    cstep = step_v[0:1, :]
    sstep = step_v[1:2, :]
    cs_v[0:1, :] = c * cstep - s * sstep
    cs_v[1:2, :] = s * cstep + c * sstep

    @pl.when(b == nb - 1)
    def _drain():
        for _ in range(_NBUF):
            pltpu.make_async_copy(
                obuf.at[pl.ds(0, _B)], o_hbm.at[pl.ds(0, _B)], osem
            ).wait()


def kernel(x, pe):
    seq_len = x.shape[-1]
    d = pe.shape[-1]
    nb = seq_len // _B

    out = pl.pallas_call(
        _rot_body,
        grid=(nb,),
        in_specs=[pl.BlockSpec(memory_space=pl.ANY)],
        out_specs=pl.BlockSpec(memory_space=pl.ANY),
        out_shape=jax.ShapeDtypeStruct((seq_len, d), jnp.float32),
        scratch_shapes=[
            pltpu.VMEM((_B, d), jnp.float32),
            pltpu.VMEM((_B, d), jnp.float32),
            pltpu.VMEM((2, d), jnp.float32),
            pltpu.VMEM((2, d), jnp.float32),
            pltpu.VMEM((_NBUF * _B, d), jnp.float32),
            pltpu.SemaphoreType.DMA,
            pltpu.SemaphoreType.DMA,
        ],
    )(pe)
    return out[None]


# speedup vs baseline: 1.4952x; 1.2002x over previous
import jax
import jax.numpy as jnp
from jax.experimental import pallas as pl
from jax.experimental.pallas import tpu as pltpu

_B = 128
_NBUF = 8


def _rot_body(pe_hbm, o_hbm, seed_v, sw_v, cs_v, step_v, obuf, osem, isem):
    b = pl.program_id(0)
    nb = pl.num_programs(0)
    d = o_hbm.shape[-1]

    @pl.when(b == 0)
    def _init():
        cp = pltpu.make_async_copy(pe_hbm.at[pl.ds(0, _B)], seed_v, isem)
        cp.start()
        cp.wait()
        seed = seed_v[...]
        lane = jax.lax.broadcasted_iota(jnp.int32, (_B, d), 1)
        even = (lane % 2) == 0
        sw_v[...] = jnp.where(
            even, jnp.roll(seed, -1, axis=1), -jnp.roll(seed, 1, axis=1)
        )
        row = seed_v[_B - 1 : _B, :]
        lane1 = jax.lax.broadcasted_iota(jnp.int32, (1, d), 1)
        even1 = (lane1 % 2) == 0
        step_v[0:1, :] = jnp.where(even1, jnp.roll(row, -1, axis=1), row)
        step_v[1:2, :] = jnp.where(even1, row, jnp.roll(row, 1, axis=1))
        cs_v[0:1, :] = jnp.ones((1, d), jnp.float32)
        cs_v[1:2, :] = jnp.zeros((1, d), jnp.float32)

    @pl.when(b >= _NBUF)
    def _credit():
        pltpu.make_async_copy(
            obuf.at[pl.ds(0, _B)], o_hbm.at[pl.ds(0, _B)], osem
        ).wait()

    slot = jax.lax.rem(b, _NBUF)
    c = cs_v[0:1, :]
    s = cs_v[1:2, :]
    obuf[pl.ds(slot * _B, _B), :] = seed_v[...] * c + sw_v[...] * s
    pltpu.make_async_copy(
        obuf.at[pl.ds(slot * _B, _B)], o_hbm.at[pl.ds(b * _B, _B)], osem
    ).start()
    cstep = step_v[0:1, :]
    sstep = step_v[1:2, :]
    cs_v[0:1, :] = c * cstep - s * sstep
    cs_v[1:2, :] = s * cstep + c * sstep

    @pl.when(b == nb - 1)
    def _drain():
        for _ in range(_NBUF):
            pltpu.make_async_copy(
                obuf.at[pl.ds(0, _B)], o_hbm.at[pl.ds(0, _B)], osem
            ).wait()


def kernel(x, pe):
    seq_len = x.shape[-1]
    d = pe.shape[-1]
    nb = seq_len // _B

    out = pl.pallas_call(
        _rot_body,
        grid=(nb,),
        in_specs=[pl.BlockSpec(memory_space=pl.ANY)],
        out_specs=pl.BlockSpec(memory_space=pl.ANY),
        out_shape=jax.ShapeDtypeStruct((seq_len, d), jnp.float32),
        scratch_shapes=[
            pltpu.VMEM((_B, d), jnp.float32),
            pltpu.VMEM((_B, d), jnp.float32),
            pltpu.VMEM((2, d), jnp.float32),
            pltpu.VMEM((2, d), jnp.float32),
            pltpu.VMEM((_NBUF * _B, d), jnp.float32),
            pltpu.SemaphoreType.DMA,
            pltpu.SemaphoreType.DMA,
        ],
    )(pe)
    return out[None]


# speedup vs baseline: 1.5014x; 1.0041x over previous
import jax
import jax.numpy as jnp
from jax.experimental import pallas as pl
from jax.experimental.pallas import tpu as pltpu

_B = 128
_NBUF = 16


def _rot_body(pe_hbm, o_hbm, seed_v, sw_v, cs_v, step_v, obuf, osem, isem):
    b = pl.program_id(0)
    nb = pl.num_programs(0)
    d = o_hbm.shape[-1]

    @pl.when(b == 0)
    def _init():
        cp = pltpu.make_async_copy(pe_hbm.at[pl.ds(0, _B)], seed_v, isem)
        cp.start()
        cp.wait()
        seed = seed_v[...]
        lane = jax.lax.broadcasted_iota(jnp.int32, (_B, d), 1)
        even = (lane % 2) == 0
        sw_v[...] = jnp.where(
            even, jnp.roll(seed, -1, axis=1), -jnp.roll(seed, 1, axis=1)
        )
        row = seed_v[_B - 1 : _B, :]
        lane1 = jax.lax.broadcasted_iota(jnp.int32, (1, d), 1)
        even1 = (lane1 % 2) == 0
        step_v[0:1, :] = jnp.where(even1, jnp.roll(row, -1, axis=1), row)
        step_v[1:2, :] = jnp.where(even1, row, jnp.roll(row, 1, axis=1))
        cs_v[0:1, :] = jnp.ones((1, d), jnp.float32)
        cs_v[1:2, :] = jnp.zeros((1, d), jnp.float32)

    @pl.when(b >= _NBUF)
    def _credit():
        pltpu.make_async_copy(
            obuf.at[pl.ds(0, _B)], o_hbm.at[pl.ds(0, _B)], osem
        ).wait()

    slot = jax.lax.rem(b, _NBUF)
    c = cs_v[0:1, :]
    s = cs_v[1:2, :]
    obuf[pl.ds(slot * _B, _B), :] = seed_v[...] * c + sw_v[...] * s
    pltpu.make_async_copy(
        obuf.at[pl.ds(slot * _B, _B)], o_hbm.at[pl.ds(b * _B, _B)], osem
    ).start()
    cstep = step_v[0:1, :]
    sstep = step_v[1:2, :]
    cs_v[0:1, :] = c * cstep - s * sstep
    cs_v[1:2, :] = s * cstep + c * sstep

    @pl.when(b == nb - 1)
    def _drain():
        for _ in range(_NBUF):
            pltpu.make_async_copy(
                obuf.at[pl.ds(0, _B)], o_hbm.at[pl.ds(0, _B)], osem
            ).wait()


def kernel(x, pe):
    seq_len = x.shape[-1]
    d = pe.shape[-1]
    nb = seq_len // _B

    out = pl.pallas_call(
        _rot_body,
        grid=(nb,),
        in_specs=[pl.BlockSpec(memory_space=pl.ANY)],
        out_specs=pl.BlockSpec(memory_space=pl.ANY),
        out_shape=jax.ShapeDtypeStruct((seq_len, d), jnp.float32),
        scratch_shapes=[
            pltpu.VMEM((_B, d), jnp.float32),
            pltpu.VMEM((_B, d), jnp.float32),
            pltpu.VMEM((2, d), jnp.float32),
            pltpu.VMEM((2, d), jnp.float32),
            pltpu.VMEM((_NBUF * _B, d), jnp.float32),
            pltpu.SemaphoreType.DMA,
            pltpu.SemaphoreType.DMA,
        ],
    )(pe)
    return out[None]
